# baseline (speedup 1.0000x reference)
import functools

import jax
import jax.numpy as jnp
from jax import lax
from jax.experimental import pallas as pl
from jax.experimental.pallas import tpu as pltpu
from jax.experimental.pallas import tpu_sc as plsc

N = 10000
NPAD = 10240
D = 128
DH = D // 2
E = 320000
NC, NS = 2, 16
NW = NC * NS
EB = 128
NBLK = 160
EPAD = NS * NBLK * EB
CHUNK = 8
NCH = NBLK // CHUNK
HYB = 2
ROWS_T = NPAD // NW
ROWS_S = NPAD // NS

_mesh = plsc.VectorSubcoreMesh(core_axis_name="c", subcore_axis_name="s")
_sc_params = pltpu.CompilerParams(needs_layout_passes=False,
                                  use_tc_tiling_on_sc=False)


@functools.partial(
    pl.kernel,
    out_type=jax.ShapeDtypeStruct((NPAD,), jnp.float32),
    mesh=_mesh,
    compiler_params=_sc_params,
    scratch_types=[
        pltpu.VMEM_SHARED((NPAD,), jnp.float32),
        pltpu.VMEM((NBLK, EB), jnp.int32),
        pltpu.VMEM((EB,), jnp.float32),
        pltpu.VMEM((ROWS_T,), jnp.float32),
        pltpu.SemaphoreType.DMA,
    ],
)
def _sc_prep(dst_hbm, dinv_hbm, deg_sp, idx_v, ones_v, row_v, dsem):
    c = lax.axis_index("c")
    s = lax.axis_index("s")
    wid = c * NS + s

    @pl.loop(0, ROWS_T, step=16)
    def _(k):
        row_v[pl.ds(k, 16)] = jnp.ones((16,), jnp.float32)

    @pl.loop(0, EB, step=16)
    def _(k):
        ones_v[pl.ds(k, 16)] = jnp.ones((16,), jnp.float32)

    pltpu.sync_copy(row_v, deg_sp.at[pl.ds(s * ROWS_S, ROWS_T)])
    pltpu.sync_copy(row_v, deg_sp.at[pl.ds(s * ROWS_S + ROWS_T, ROWS_T)])
    plsc.subcore_barrier()

    pltpu.sync_copy(dst_hbm.at[s], idx_v)

    @pl.loop(0, NBLK)
    def _(j):
        pltpu.async_copy(ones_v, deg_sp.at[idx_v.at[j]], dsem, add=True)

    @pl.loop(0, NBLK)
    def _(j):
        pltpu.make_async_copy(ones_v, deg_sp.at[idx_v.at[0]], dsem).wait()

    plsc.subcore_barrier()

    pltpu.sync_copy(deg_sp.at[pl.ds(wid * ROWS_T, ROWS_T)], row_v)

    @pl.loop(0, ROWS_T, step=16)
    def _(k):
        v = row_v[pl.ds(k, 16)]
        yi = jnp.int32(0x5F3759DF) - (plsc.bitcast(v, jnp.int32) >> 1)
        y = plsc.bitcast(yi, jnp.float32)
        for _ in range(3):
            y = y * (1.5 - 0.5 * v * y * y)
        row_v[pl.ds(k, 16)] = y

    pltpu.sync_copy(row_v, dinv_hbm.at[pl.ds(wid * ROWS_T, ROWS_T)])


@functools.partial(
    pl.kernel,
    out_type=jax.ShapeDtypeStruct((NC, NPAD, DH), jnp.float32),
    mesh=_mesh,
    compiler_params=_sc_params,
    scratch_types=[
        pltpu.VMEM_SHARED((NPAD, DH), jnp.float32),
        pltpu.VMEM_SHARED((NPAD, DH), jnp.float32),
        [pltpu.VMEM((CHUNK, EB), jnp.int32)] * 2,
        [pltpu.VMEM((CHUNK, EB), jnp.int32)] * 2,
        [pltpu.VMEM((EB, DH), jnp.float32)] * 4,
        [pltpu.SemaphoreType.DMA] * 2,
        [pltpu.SemaphoreType.DMA] * 4,
        [pltpu.SemaphoreType.DMA] * 4,
    ],
)
def _sc_agg(g_hbm, src_hbm, dst_hbm, s_hbm, g_sp, acc_sp, srcc, dstc,
            rows, isem, gsem, ssem):
    c = lax.axis_index("c")
    s = lax.axis_index("s")

    pltpu.async_copy(src_hbm.at[s, pl.ds(0, CHUNK)], srcc[0], isem[0])
    pltpu.async_copy(dst_hbm.at[s, pl.ds(0, CHUNK)], dstc[0], isem[0])

    @pl.loop(0, EB)
    def _(r):
        @pl.loop(0, DH, step=16)
        def _(k):
            rows[0][r, pl.ds(k, 16)] = jnp.zeros((16,), jnp.float32)

    @pl.loop(0, ROWS_S, step=EB)
    def _(r):
        pltpu.sync_copy(rows[0], acc_sp.at[pl.ds(s * ROWS_S + r, EB)])

    pltpu.sync_copy(g_hbm.at[c, pl.ds(s * ROWS_S, ROWS_S)],
                    g_sp.at[pl.ds(s * ROWS_S, ROWS_S)])
    plsc.subcore_barrier()

    def _wait_idx(st):
        pltpu.make_async_copy(src_hbm.at[s, pl.ds(0, CHUNK)], srcc[st],
                              isem[st]).wait()
        pltpu.make_async_copy(dst_hbm.at[s, pl.ds(0, CHUNK)], dstc[st],
                              isem[st]).wait()

    _wait_idx(0)
    for k in range(3):
        gsrc = g_hbm.at[c] if k < HYB else g_sp
        pltpu.async_copy(gsrc.at[srcc[0].at[k]], rows[k], gsem[k])

    def _half(ch, st):
        nst = 1 - st

        @pl.when(ch + 1 < NCH)
        def _():
            pltpu.async_copy(src_hbm.at[s, pl.ds((ch + 1) * CHUNK, CHUNK)],
                             srcc[nst], isem[nst])
            pltpu.async_copy(dst_hbm.at[s, pl.ds((ch + 1) * CHUNK, CHUNK)],
                             dstc[nst], isem[nst])

        for i in range(CHUNK):
            b = ch * CHUNK + i
            k = i % 4
            kn = (i + 3) % 4

            pltpu.make_async_copy(g_sp.at[srcc[st].at[i]], rows[k],
                                  gsem[k]).wait()
            pltpu.async_copy(rows[k], acc_sp.at[dstc[st].at[i]], ssem[k],
                             add=True)

            if i < CHUNK - 3:
                @pl.when(b >= 1)
                def _():
                    pltpu.make_async_copy(rows[kn],
                                          acc_sp.at[dstc[st].at[i]],
                                          ssem[kn]).wait()
                pltpu.async_copy(g_sp.at[srcc[st].at[i + 3]], rows[kn],
                                 gsem[kn])
            else:
                pos = i - (CHUNK - 3)
                gsrc = g_hbm.at[c] if pos < HYB else g_sp

                @pl.when(ch + 1 < NCH)
                def _():
                    if i == CHUNK - 3:
                        _wait_idx(nst)
                    pltpu.make_async_copy(rows[kn],
                                          acc_sp.at[dstc[st].at[i]],
                                          ssem[kn]).wait()
                    pltpu.async_copy(gsrc.at[srcc[nst].at[pos]],
                                     rows[kn], gsem[kn])

    @pl.loop(0, NCH, step=2)
    def _(ch):
        _half(ch, 0)
        _half(ch + 1, 1)

    for k in range(4):
        pltpu.make_async_copy(rows[k], acc_sp.at[dstc[1].at[k]],
                              ssem[k]).wait()

    plsc.subcore_barrier()
    pltpu.sync_copy(acc_sp.at[pl.ds(s * ROWS_S, ROWS_S)],
                    s_hbm.at[c, pl.ds(s * ROWS_S, ROWS_S)])


_RB = 1280
_row_spec = pl.BlockSpec((_RB, D), lambda i: (i, 0))
_pair_spec = pl.BlockSpec((NC, _RB, DH), lambda i: (0, i, 0))
_col_spec = pl.BlockSpec((_RB, 1), lambda i: (i, 0))
_w_spec = pl.BlockSpec((D, D), lambda i: (0, 0))
_b_spec = pl.BlockSpec((1, D), lambda i: (0, 0))
_pair_shape = jax.ShapeDtypeStruct((NC, NPAD, DH), jnp.float32)


def _split_store(o_ref, val):
    o_ref[0] = val[:, :DH]
    o_ref[1] = val[:, DH:]


def _tc_project_body(x_ref, w_ref, dinv_ref, g_ref):
    _split_store(g_ref, dinv_ref[...] * jnp.dot(
        x_ref[...], w_ref[...], preferred_element_type=jnp.float32))


def _tc_project(x, w, dinv_col):
    return pl.pallas_call(
        _tc_project_body,
        grid=(NPAD // _RB,),
        in_specs=[_row_spec, _w_spec, _col_spec],
        out_specs=_pair_spec,
        out_shape=_pair_shape,
    )(x, w, dinv_col)


def _agg_concat(s_ref, g_ref):
    return jnp.concatenate(
        [s_ref[0] + g_ref[0], s_ref[1] + g_ref[1]], axis=1)


def _tc_layer_body(s_ref, g_ref, dinv_ref, b_ref, w_ref, o_ref):
    u = jax.nn.relu(dinv_ref[...] * _agg_concat(s_ref, g_ref) + b_ref[...])
    _split_store(o_ref, dinv_ref[...] * jnp.dot(
        u, w_ref[...], preferred_element_type=jnp.float32))


def _tc_layer(s, g, dinv_col, b, w):
    return pl.pallas_call(
        _tc_layer_body,
        grid=(NPAD // _RB,),
        in_specs=[_pair_spec, _pair_spec, _col_spec, _b_spec, _w_spec],
        out_specs=_pair_spec,
        out_shape=_pair_shape,
    )(s, g, dinv_col, b, w)


def _tc_final_body(s_ref, g_ref, dinv_ref, b_ref, o_ref):
    o_ref[...] = jax.nn.relu(
        dinv_ref[...] * _agg_concat(s_ref, g_ref) + b_ref[...])


def _tc_final(s, g, dinv_col, b):
    return pl.pallas_call(
        _tc_final_body,
        grid=(NPAD // _RB,),
        in_specs=[_pair_spec, _pair_spec, _col_spec, _b_spec],
        out_specs=_row_spec,
        out_shape=jax.ShapeDtypeStruct((NPAD, D), jnp.float32),
    )(s, g, dinv_col, b)


def kernel(x, edge_index, W1, b1, W2, b2, W3, b3):
    src = edge_index[0].astype(jnp.int32)
    dst = edge_index[1].astype(jnp.int32)
    src_p = jnp.concatenate(
        [src, jnp.zeros((EPAD - E,), jnp.int32)]).reshape(NS, NBLK, EB)
    dst_p = jnp.concatenate(
        [dst, jnp.full((EPAD - E,), N, jnp.int32)]).reshape(NS, NBLK, EB)
    x_p = jnp.pad(x, ((0, NPAD - N), (0, 0)))

    dinv = _sc_prep(dst_p)
    dinv_col = dinv.reshape(NPAD, 1)

    g = _tc_project(x_p, W1, dinv_col)
    S = _sc_agg(g, src_p, dst_p)
    g = _tc_layer(S, g, dinv_col, b1.reshape(1, D), W2)
    S = _sc_agg(g, src_p, dst_p)
    g = _tc_layer(S, g, dinv_col, b2.reshape(1, D), W3)
    S = _sc_agg(g, src_p, dst_p)
    out = _tc_final(S, g, dinv_col, b3.reshape(1, D))
    return out[:N]


# speedup vs baseline: 1.2769x; 1.2769x over previous
import functools

import jax
import jax.numpy as jnp
from jax import lax
from jax.experimental import pallas as pl
from jax.experimental.pallas import tpu as pltpu
from jax.experimental.pallas import tpu_sc as plsc

N = 10000
NPAD = 10240
D = 128
DH = D // 2
E = 320000
NC, NS = 2, 16
NW = NC * NS
EB = 128
NBLK = 160
EPAD = NS * NBLK * EB
CHUNK = 8
NCH = NBLK // CHUNK
HYB = 0
ROWS_T = NPAD // NW
ROWS_S = NPAD // NS

_mesh = plsc.VectorSubcoreMesh(core_axis_name="c", subcore_axis_name="s")
_sc_params = pltpu.CompilerParams(needs_layout_passes=False,
                                  use_tc_tiling_on_sc=False)


@functools.partial(
    pl.kernel,
    out_type=jax.ShapeDtypeStruct((NPAD,), jnp.float32),
    mesh=_mesh,
    compiler_params=_sc_params,
    scratch_types=[
        pltpu.VMEM_SHARED((NPAD,), jnp.float32),
        pltpu.VMEM((NBLK, EB), jnp.int32),
        pltpu.VMEM((EB,), jnp.float32),
        pltpu.VMEM((ROWS_T,), jnp.float32),
        pltpu.SemaphoreType.DMA,
    ],
)
def _sc_prep(dst_hbm, dinv_hbm, deg_sp, idx_v, ones_v, row_v, dsem):
    c = lax.axis_index("c")
    s = lax.axis_index("s")
    wid = c * NS + s

    @pl.loop(0, ROWS_T, step=16)
    def _(k):
        row_v[pl.ds(k, 16)] = jnp.ones((16,), jnp.float32)

    @pl.loop(0, EB, step=16)
    def _(k):
        ones_v[pl.ds(k, 16)] = jnp.ones((16,), jnp.float32)

    pltpu.sync_copy(row_v, deg_sp.at[pl.ds(s * ROWS_S, ROWS_T)])
    pltpu.sync_copy(row_v, deg_sp.at[pl.ds(s * ROWS_S + ROWS_T, ROWS_T)])
    plsc.subcore_barrier()

    pltpu.sync_copy(dst_hbm.at[s], idx_v)

    @pl.loop(0, NBLK)
    def _(j):
        pltpu.async_copy(ones_v, deg_sp.at[idx_v.at[j]], dsem, add=True)

    @pl.loop(0, NBLK)
    def _(j):
        pltpu.make_async_copy(ones_v, deg_sp.at[idx_v.at[0]], dsem).wait()

    plsc.subcore_barrier()

    pltpu.sync_copy(deg_sp.at[pl.ds(wid * ROWS_T, ROWS_T)], row_v)

    @pl.loop(0, ROWS_T, step=16)
    def _(k):
        v = row_v[pl.ds(k, 16)]
        yi = jnp.int32(0x5F3759DF) - (plsc.bitcast(v, jnp.int32) >> 1)
        y = plsc.bitcast(yi, jnp.float32)
        for _ in range(3):
            y = y * (1.5 - 0.5 * v * y * y)
        row_v[pl.ds(k, 16)] = y

    pltpu.sync_copy(row_v, dinv_hbm.at[pl.ds(wid * ROWS_T, ROWS_T)])


@functools.partial(
    pl.kernel,
    out_type=jax.ShapeDtypeStruct((NC, NPAD, DH), jnp.float32),
    mesh=_mesh,
    compiler_params=_sc_params,
    scratch_types=[
        pltpu.VMEM_SHARED((NPAD, DH), jnp.float32),
        pltpu.VMEM_SHARED((NPAD, DH), jnp.float32),
        [pltpu.VMEM((CHUNK, EB), jnp.int32)] * 2,
        [pltpu.VMEM((CHUNK, EB), jnp.int32)] * 2,
        [pltpu.VMEM((EB, DH), jnp.float32)] * 4,
        [pltpu.SemaphoreType.DMA] * 2,
        [pltpu.SemaphoreType.DMA] * 4,
        [pltpu.SemaphoreType.DMA] * 4,
    ],
)
def _sc_agg(g_hbm, src_hbm, dst_hbm, s_hbm, g_sp, acc_sp, srcc, dstc,
            rows, isem, gsem, ssem):
    c = lax.axis_index("c")
    s = lax.axis_index("s")

    pltpu.async_copy(src_hbm.at[s, pl.ds(0, CHUNK)], srcc[0], isem[0])
    pltpu.async_copy(dst_hbm.at[s, pl.ds(0, CHUNK)], dstc[0], isem[0])

    @pl.loop(0, EB)
    def _(r):
        @pl.loop(0, DH, step=16)
        def _(k):
            rows[0][r, pl.ds(k, 16)] = jnp.zeros((16,), jnp.float32)

    @pl.loop(0, ROWS_S, step=EB)
    def _(r):
        pltpu.sync_copy(rows[0], acc_sp.at[pl.ds(s * ROWS_S + r, EB)])

    pltpu.sync_copy(g_hbm.at[c, pl.ds(s * ROWS_S, ROWS_S)],
                    g_sp.at[pl.ds(s * ROWS_S, ROWS_S)])
    plsc.subcore_barrier()

    def _wait_idx(st):
        pltpu.make_async_copy(src_hbm.at[s, pl.ds(0, CHUNK)], srcc[st],
                              isem[st]).wait()
        pltpu.make_async_copy(dst_hbm.at[s, pl.ds(0, CHUNK)], dstc[st],
                              isem[st]).wait()

    _wait_idx(0)
    for k in range(3):
        gsrc = g_hbm.at[c] if k < HYB else g_sp
        pltpu.async_copy(gsrc.at[srcc[0].at[k]], rows[k], gsem[k])

    def _half(ch, st):
        nst = 1 - st

        @pl.when(ch + 1 < NCH)
        def _():
            pltpu.async_copy(src_hbm.at[s, pl.ds((ch + 1) * CHUNK, CHUNK)],
                             srcc[nst], isem[nst])
            pltpu.async_copy(dst_hbm.at[s, pl.ds((ch + 1) * CHUNK, CHUNK)],
                             dstc[nst], isem[nst])

        for i in range(CHUNK):
            b = ch * CHUNK + i
            k = i % 4
            kn = (i + 3) % 4

            pltpu.make_async_copy(g_sp.at[srcc[st].at[i]], rows[k],
                                  gsem[k]).wait()
            pltpu.async_copy(rows[k], acc_sp.at[dstc[st].at[i]], ssem[k],
                             add=True)

            if i < CHUNK - 3:
                @pl.when(b >= 1)
                def _():
                    pltpu.make_async_copy(rows[kn],
                                          acc_sp.at[dstc[st].at[i]],
                                          ssem[kn]).wait()
                pltpu.async_copy(g_sp.at[srcc[st].at[i + 3]], rows[kn],
                                 gsem[kn])
            else:
                pos = i - (CHUNK - 3)
                gsrc = g_hbm.at[c] if pos < HYB else g_sp

                @pl.when(ch + 1 < NCH)
                def _():
                    if i == CHUNK - 3:
                        _wait_idx(nst)
                    pltpu.make_async_copy(rows[kn],
                                          acc_sp.at[dstc[st].at[i]],
                                          ssem[kn]).wait()
                    pltpu.async_copy(gsrc.at[srcc[nst].at[pos]],
                                     rows[kn], gsem[kn])

    @pl.loop(0, NCH, step=2)
    def _(ch):
        _half(ch, 0)
        _half(ch + 1, 1)

    for k in range(4):
        pltpu.make_async_copy(rows[k], acc_sp.at[dstc[1].at[k]],
                              ssem[k]).wait()

    plsc.subcore_barrier()
    pltpu.sync_copy(acc_sp.at[pl.ds(s * ROWS_S, ROWS_S)],
                    s_hbm.at[c, pl.ds(s * ROWS_S, ROWS_S)])


_RB = 1280
_row_spec = pl.BlockSpec((_RB, D), lambda i: (i, 0))
_pair_spec = pl.BlockSpec((NC, _RB, DH), lambda i: (0, i, 0))
_col_spec = pl.BlockSpec((_RB, 1), lambda i: (i, 0))
_w_spec = pl.BlockSpec((D, D), lambda i: (0, 0))
_b_spec = pl.BlockSpec((1, D), lambda i: (0, 0))
_pair_shape = jax.ShapeDtypeStruct((NC, NPAD, DH), jnp.float32)


def _split_store(o_ref, val):
    o_ref[0] = val[:, :DH]
    o_ref[1] = val[:, DH:]


def _tc_project_body(x_ref, w_ref, dinv_ref, g_ref):
    _split_store(g_ref, dinv_ref[...] * jnp.dot(
        x_ref[...], w_ref[...], preferred_element_type=jnp.float32))


def _tc_project(x, w, dinv_col):
    return pl.pallas_call(
        _tc_project_body,
        grid=(NPAD // _RB,),
        in_specs=[_row_spec, _w_spec, _col_spec],
        out_specs=_pair_spec,
        out_shape=_pair_shape,
    )(x, w, dinv_col)


def _agg_concat(s_ref, g_ref):
    return jnp.concatenate(
        [s_ref[0] + g_ref[0], s_ref[1] + g_ref[1]], axis=1)


def _tc_layer_body(s_ref, g_ref, dinv_ref, b_ref, w_ref, o_ref):
    u = jax.nn.relu(dinv_ref[...] * _agg_concat(s_ref, g_ref) + b_ref[...])
    _split_store(o_ref, dinv_ref[...] * jnp.dot(
        u, w_ref[...], preferred_element_type=jnp.float32))


def _tc_layer(s, g, dinv_col, b, w):
    return pl.pallas_call(
        _tc_layer_body,
        grid=(NPAD // _RB,),
        in_specs=[_pair_spec, _pair_spec, _col_spec, _b_spec, _w_spec],
        out_specs=_pair_spec,
        out_shape=_pair_shape,
    )(s, g, dinv_col, b, w)


def _tc_final_body(s_ref, g_ref, dinv_ref, b_ref, o_ref):
    o_ref[...] = jax.nn.relu(
        dinv_ref[...] * _agg_concat(s_ref, g_ref) + b_ref[...])


def _tc_final(s, g, dinv_col, b):
    return pl.pallas_call(
        _tc_final_body,
        grid=(NPAD // _RB,),
        in_specs=[_pair_spec, _pair_spec, _col_spec, _b_spec],
        out_specs=_row_spec,
        out_shape=jax.ShapeDtypeStruct((NPAD, D), jnp.float32),
    )(s, g, dinv_col, b)


def kernel(x, edge_index, W1, b1, W2, b2, W3, b3):
    src = edge_index[0].astype(jnp.int32)
    dst = edge_index[1].astype(jnp.int32)
    src_p = jnp.concatenate(
        [src, jnp.zeros((EPAD - E,), jnp.int32)]).reshape(NS, NBLK, EB)
    dst_p = jnp.concatenate(
        [dst, jnp.full((EPAD - E,), N, jnp.int32)]).reshape(NS, NBLK, EB)
    x_p = jnp.pad(x, ((0, NPAD - N), (0, 0)))

    dinv = _sc_prep(dst_p)
    dinv_col = dinv.reshape(NPAD, 1)

    g = _tc_project(x_p, W1, dinv_col)
    S = _sc_agg(g, src_p, dst_p)
    g = _tc_layer(S, g, dinv_col, b1.reshape(1, D), W2)
    S = _sc_agg(g, src_p, dst_p)
    g = _tc_layer(S, g, dinv_col, b2.reshape(1, D), W3)
    S = _sc_agg(g, src_p, dst_p)
    out = _tc_final(S, g, dinv_col, b3.reshape(1, D))
    return out[:N]
